# baseline (speedup 1.0000x reference)
import functools

import jax
import jax.numpy as jnp
from jax import lax
from jax.experimental import pallas as pl
from jax.experimental.pallas import tpu as pltpu
from jax.experimental.pallas import tpu_sc as plsc

BS = 1024
D = 64
N_CAND = 501
NT = 512
NW = 32
B_PER = BS // NW
CHUNK = 128
NQ = NT // CHUNK
NLANE = 16


def _sc_body(tabR, tabI, relR, relI, sub_h, rel_h, ent_h, dR_h, dI_h, out_h,
             sub_v, rel_v, esR, esI, rrR, rrI, dR, dI, aR, aI, ndR,
             idx_v, rowsR, rowsI, scores, tile, sem0, sem1, sem2):
    wid = lax.axis_index("s") * 2 + lax.axis_index("c")
    base_b = wid * B_PER

    pltpu.sync_copy(sub_h.at[pl.ds(base_b, B_PER)], sub_v)
    pltpu.sync_copy(rel_h.at[pl.ds(base_b, B_PER)], rel_v)
    pltpu.sync_copy(dR_h.at[pl.ds(base_b, B_PER)], dR)
    pltpu.sync_copy(dI_h.at[pl.ds(base_b, B_PER)], dI)
    pltpu.async_copy(tabR.at[sub_v], esR, sem0).wait()
    pltpu.async_copy(tabI.at[sub_v], esI, sem0).wait()
    pltpu.async_copy(relR.at[rel_v], rrR, sem0).wait()
    pltpu.async_copy(relI.at[rel_v], rrI, sem0).wait()

    def prep(i, carry):
        for c in range(D // NLANE):
            s = pl.ds(c * NLANE, NLANE)
            er = esR[i, s]
            ei = esI[i, s]
            dr = dR[i, s]
            di = dI[i, s]
            aR[i, s] = er * dr - ei * di + rrR[i, s]
            aI[i, s] = er * di + ei * dr + rrI[i, s]
            ndR[i, s] = -dr
        return carry

    lax.fori_loop(0, B_PER, prep, 0)

    lane = lax.iota(jnp.int32, NLANE)

    def step(st, carry):
        b_local = lax.shift_right_logical(st, 2)
        q = lax.bitwise_and(st, NQ - 1)
        b = base_b + b_local
        pltpu.sync_copy(ent_h.at[b, pl.ds(q * CHUNK, CHUNK)], idx_v)
        cR = pltpu.async_copy(tabR.at[idx_v], rowsR, sem1)
        cI = pltpu.async_copy(tabI.at[idx_v], rowsI, sem2)
        cR.wait()
        cI.wait()

        nc = D // NLANE
        ars = [aR[b_local, pl.ds(c * NLANE, NLANE)] for c in range(nc)]
        ais = [aI[b_local, pl.ds(c * NLANE, NLANE)] for c in range(nc)]
        ndrs = [ndR[b_local, pl.ds(c * NLANE, NLANE)] for c in range(nc)]
        drs = [dR[b_local, pl.ds(c * NLANE, NLANE)] for c in range(nc)]
        dis = [dI[b_local, pl.ds(c * NLANE, NLANE)] for c in range(nc)]

        def group(h, gcarry):
            for jj in range(NLANE):
                j = h * NLANE + jj
                acc = jnp.zeros((NLANE,), jnp.float32)
                for c in range(nc):
                    s = pl.ds(c * NLANE, NLANE)
                    er = rowsR[j, s]
                    ei = rowsI[j, s]
                    t1 = er * ndrs[c] + ei * dis[c] + ars[c]
                    t2 = er * dis[c] + ei * drs[c] + ais[c]
                    acc = acc + jnp.abs(t1) + jnp.abs(t2)
                tile[jj, :] = acc
            vec = jnp.zeros((NLANE,), jnp.float32)
            for k in range(NLANE):
                vec = vec + plsc.load_gather(
                    tile, [lane, jnp.full((NLANE,), k, jnp.int32)])
            scores[pl.ds(q * CHUNK + h * NLANE, NLANE)] = vec
            return gcarry

        lax.fori_loop(0, CHUNK // NLANE, group, 0)

        @pl.when(q == NQ - 1)
        def _():
            pltpu.sync_copy(scores, out_h.at[b])

        return carry

    lax.fori_loop(0, B_PER * NQ, step, 0)


def _sc_scores(tabR, tabI, relR, relI, sub, rel, ent, d_real, d_img):
    mesh = plsc.VectorSubcoreMesh(core_axis_name="c", subcore_axis_name="s")
    f = pl.kernel(
        _sc_body,
        out_type=jax.ShapeDtypeStruct((BS, NT), jnp.float32),
        mesh=mesh,
        compiler_params=pltpu.CompilerParams(
            needs_layout_passes=False, use_tc_tiling_on_sc=False),
        scratch_types=[
            pltpu.VMEM((B_PER,), jnp.int32),
            pltpu.VMEM((B_PER,), jnp.int32),
            pltpu.VMEM((B_PER, D), jnp.float32),
            pltpu.VMEM((B_PER, D), jnp.float32),
            pltpu.VMEM((B_PER, D), jnp.float32),
            pltpu.VMEM((B_PER, D), jnp.float32),
            pltpu.VMEM((B_PER, D), jnp.float32),
            pltpu.VMEM((B_PER, D), jnp.float32),
            pltpu.VMEM((B_PER, D), jnp.float32),
            pltpu.VMEM((B_PER, D), jnp.float32),
            pltpu.VMEM((B_PER, D), jnp.float32),
            pltpu.VMEM((CHUNK,), jnp.int32),
            pltpu.VMEM((CHUNK, D), jnp.float32),
            pltpu.VMEM((CHUNK, D), jnp.float32),
            pltpu.VMEM((NT,), jnp.float32),
            pltpu.VMEM((NLANE, NLANE), jnp.float32),
            pltpu.SemaphoreType.DMA,
            pltpu.SemaphoreType.DMA,
            pltpu.SemaphoreType.DMA,
        ],
    )
    return f(tabR, tabI, relR, relI, sub, rel, ent, d_real, d_img)


def _trig_body(day_ref, w1_ref, w2_ref, dr_ref, di_ref):
    day = day_ref[...]
    dr_ref[...] = jnp.cos(w2_ref[...] * day)
    di_ref[...] = jnp.sin(w1_ref[...] * day)


def _loss_body(x_ref, out_ref):
    x = x_ref[...]
    col = lax.broadcasted_iota(jnp.int32, (BS, NT), 1)
    valid = col < N_CAND
    xm = jnp.where(valid, x, -jnp.inf)
    m = jnp.max(xm, axis=1, keepdims=True)
    ssum = jnp.sum(jnp.where(valid, jnp.exp(xm - m), 0.0), axis=1,
                   keepdims=True)
    lse = m + jnp.log(ssum)
    pos = jnp.sum(jnp.where(col == 0, x, 0.0), axis=1, keepdims=True)
    val = jnp.sum(lse - pos) * (1.0 / BS)
    out_ref[...] = jnp.full((1, 1), val, jnp.float32)


def kernel(sub, rel, obj, year, month, day, neg, emb_E_real, emb_E_img,
           emb_R_real, emb_R_img, w1, w2):
    ent = jnp.concatenate([obj[:, None], neg], axis=1).astype(jnp.int32)
    ent = jnp.pad(ent, ((0, 0), (0, NT - N_CAND)))

    d_real, d_img = pl.pallas_call(
        _trig_body,
        out_shape=[
            jax.ShapeDtypeStruct((BS, D), jnp.float32),
            jax.ShapeDtypeStruct((BS, D), jnp.float32),
        ],
    )(day.reshape(BS, 1), w1.reshape(1, D), w2.reshape(1, D))

    scores = _sc_scores(emb_E_real, emb_E_img, emb_R_real, emb_R_img,
                        sub.astype(jnp.int32), rel.astype(jnp.int32),
                        ent, d_real, d_img)

    loss = pl.pallas_call(
        _loss_body,
        out_shape=jax.ShapeDtypeStruct((1, 1), jnp.float32),
    )(scores)
    return loss[0, 0]


# speedup vs baseline: 1.1294x; 1.1294x over previous
import jax
import jax.numpy as jnp
from jax import lax
from jax.experimental import pallas as pl
from jax.experimental.pallas import tpu as pltpu
from jax.experimental.pallas import tpu_sc as plsc

BS = 1024
D = 64
N_CAND = 501
NT = 512
NW = 32
B_PER = BS // NW
CHUNK = 128
NQ = NT // CHUNK
NSTEP = B_PER * NQ
NLANE = 16
NC = D // NLANE


def _sc_body(tabE, tabR, sub_h, rel_h, ent_h, dR_h, dI_h, out_h,
             sub_v, rel_v, es, rr, dR, dI, aR, aI, ndR,
             idx_v, rows, scores, tile, sem0, semg0, semg1):
    wid = lax.axis_index("s") * 2 + lax.axis_index("c")
    base_b = wid * B_PER

    pltpu.sync_copy(sub_h.at[pl.ds(base_b, B_PER)], sub_v)
    pltpu.sync_copy(rel_h.at[pl.ds(base_b, B_PER)], rel_v)
    pltpu.sync_copy(dR_h.at[pl.ds(base_b, B_PER)], dR)
    pltpu.sync_copy(dI_h.at[pl.ds(base_b, B_PER)], dI)
    pltpu.async_copy(tabE.at[sub_v], es, sem0).wait()
    pltpu.async_copy(tabR.at[rel_v], rr, sem0).wait()

    def prep(i, carry):
        for c in range(NC):
            s = pl.ds(c * NLANE, NLANE)
            si = pl.ds(D + c * NLANE, NLANE)
            er = es[i, s]
            ei = es[i, si]
            dr = dR[i, s]
            di = dI[i, s]
            aR[i, s] = er * dr - ei * di + rr[i, s]
            aI[i, s] = er * di + ei * dr + rr[i, si]
            ndR[i, s] = -dr
        return carry

    lax.fori_loop(0, B_PER, prep, 0)

    lane = lax.iota(jnp.int32, NLANE)
    bufs = ((rows.at[0], idx_v.at[0], semg0), (rows.at[1], idx_v.at[1], semg1))

    def issue(step, db):
        b_local = lax.shift_right_logical(step, 2)
        q = lax.bitwise_and(step, NQ - 1)
        rbuf, ibuf, sem = bufs[db]
        pltpu.sync_copy(ent_h.at[base_b + b_local, pl.ds(q * CHUNK, CHUNK)],
                        ibuf)
        pltpu.async_copy(tabE.at[ibuf], rbuf, sem)

    issue(0, 0)
    issue(1, 1)

    def compute(step, db):
        b_local = lax.shift_right_logical(step, 2)
        q = lax.bitwise_and(step, NQ - 1)
        rbuf, ibuf, sem = bufs[db]
        pltpu.make_async_copy(tabE.at[ibuf], rbuf, sem).wait()

        ars = [aR[b_local, pl.ds(c * NLANE, NLANE)] for c in range(NC)]
        ais = [aI[b_local, pl.ds(c * NLANE, NLANE)] for c in range(NC)]
        ndrs = [ndR[b_local, pl.ds(c * NLANE, NLANE)] for c in range(NC)]
        drs = [dR[b_local, pl.ds(c * NLANE, NLANE)] for c in range(NC)]
        dis = [dI[b_local, pl.ds(c * NLANE, NLANE)] for c in range(NC)]

        def group(h, gcarry):
            for jj in range(NLANE):
                j = h * NLANE + jj
                acc = jnp.zeros((NLANE,), jnp.float32)
                for c in range(NC):
                    er = rbuf[j, pl.ds(c * NLANE, NLANE)]
                    ei = rbuf[j, pl.ds(D + c * NLANE, NLANE)]
                    t1 = er * ndrs[c] + ei * dis[c] + ars[c]
                    t2 = er * dis[c] + ei * drs[c] + ais[c]
                    acc = acc + jnp.abs(t1) + jnp.abs(t2)
                tile[jj, :] = acc
            vec = jnp.zeros((NLANE,), jnp.float32)
            for k in range(NLANE):
                vec = vec + plsc.load_gather(
                    tile, [lane, jnp.full((NLANE,), k, jnp.int32)])
            scores[pl.ds(q * CHUNK + h * NLANE, NLANE)] = vec
            return gcarry

        lax.fori_loop(0, CHUNK // NLANE, group, 0)

        @pl.when(q == NQ - 1)
        def _():
            pltpu.sync_copy(scores, out_h.at[base_b + b_local])

    def step_pair(t, carry):
        for db in range(2):
            s = 2 * t + db
            compute(s, db)

            @pl.when(s + 2 < NSTEP)
            def _():
                issue(s + 2, db)

        return carry

    lax.fori_loop(0, NSTEP // 2, step_pair, 0)


def _sc_scores(tabE, tabR, sub, rel, ent, d_real, d_img):
    mesh = plsc.VectorSubcoreMesh(core_axis_name="c", subcore_axis_name="s")
    f = pl.kernel(
        _sc_body,
        out_type=jax.ShapeDtypeStruct((BS, NT), jnp.float32),
        mesh=mesh,
        compiler_params=pltpu.CompilerParams(
            needs_layout_passes=False, use_tc_tiling_on_sc=False),
        scratch_types=[
            pltpu.VMEM((B_PER,), jnp.int32),
            pltpu.VMEM((B_PER,), jnp.int32),
            pltpu.VMEM((B_PER, 2 * D), jnp.float32),
            pltpu.VMEM((B_PER, 2 * D), jnp.float32),
            pltpu.VMEM((B_PER, D), jnp.float32),
            pltpu.VMEM((B_PER, D), jnp.float32),
            pltpu.VMEM((B_PER, D), jnp.float32),
            pltpu.VMEM((B_PER, D), jnp.float32),
            pltpu.VMEM((B_PER, D), jnp.float32),
            pltpu.VMEM((2, CHUNK), jnp.int32),
            pltpu.VMEM((2, CHUNK, 2 * D), jnp.float32),
            pltpu.VMEM((NT,), jnp.float32),
            pltpu.VMEM((NLANE, NLANE), jnp.float32),
            pltpu.SemaphoreType.DMA,
            pltpu.SemaphoreType.DMA,
            pltpu.SemaphoreType.DMA,
        ],
    )
    return f(tabE, tabR, sub, rel, ent, d_real, d_img)


def _trig_body(day_ref, w1_ref, w2_ref, dr_ref, di_ref):
    day = day_ref[...]
    dr_ref[...] = jnp.cos(w2_ref[...] * day)
    di_ref[...] = jnp.sin(w1_ref[...] * day)


def _loss_body(x_ref, out_ref):
    x = x_ref[...]
    col = lax.broadcasted_iota(jnp.int32, (BS, NT), 1)
    valid = col < N_CAND
    xm = jnp.where(valid, x, -jnp.inf)
    m = jnp.max(xm, axis=1, keepdims=True)
    ssum = jnp.sum(jnp.where(valid, jnp.exp(xm - m), 0.0), axis=1,
                   keepdims=True)
    lse = m + jnp.log(ssum)
    pos = jnp.sum(jnp.where(col == 0, x, 0.0), axis=1, keepdims=True)
    val = jnp.sum(lse - pos) * (1.0 / BS)
    out_ref[...] = jnp.full((1, 1), val, jnp.float32)


def kernel(sub, rel, obj, year, month, day, neg, emb_E_real, emb_E_img,
           emb_R_real, emb_R_img, w1, w2):
    ent = jnp.concatenate([obj[:, None], neg], axis=1).astype(jnp.int32)
    ent = jnp.pad(ent, ((0, 0), (0, NT - N_CAND)))

    tabE = jnp.concatenate([emb_E_real, emb_E_img], axis=1)
    tabR = jnp.concatenate([emb_R_real, emb_R_img], axis=1)

    d_real, d_img = pl.pallas_call(
        _trig_body,
        out_shape=[
            jax.ShapeDtypeStruct((BS, D), jnp.float32),
            jax.ShapeDtypeStruct((BS, D), jnp.float32),
        ],
    )(day.reshape(BS, 1), w1.reshape(1, D), w2.reshape(1, D))

    scores = _sc_scores(tabE, tabR, sub.astype(jnp.int32),
                        rel.astype(jnp.int32), ent, d_real, d_img)

    loss = pl.pallas_call(
        _loss_body,
        out_shape=jax.ShapeDtypeStruct((1, 1), jnp.float32),
    )(scores)
    return loss[0, 0]


# speedup vs baseline: 1.1333x; 1.0034x over previous
import jax
import jax.numpy as jnp
from jax import lax
from jax.experimental import pallas as pl
from jax.experimental.pallas import tpu as pltpu
from jax.experimental.pallas import tpu_sc as plsc

BS = 1024
D = 64
N_CAND = 501
NT = 512
NW = 32
B_PER = BS // NW
CHUNK = 128
NQ = NT // CHUNK
NSTEP = B_PER * NQ
NLANE = 16
NC = D // NLANE


def _sc_body(tabE, tabR, sub_h, rel_h, ent_h, dR_h, dI_h, out_h,
             sub_v, rel_v, es, rr, dR, dI, aR, aI, ndR,
             idx_v, rows, scores, tile, sem0, semg0, semg1):
    wid = lax.axis_index("s") * 2 + lax.axis_index("c")
    base_b = wid * B_PER

    pltpu.sync_copy(sub_h.at[pl.ds(base_b, B_PER)], sub_v)
    pltpu.sync_copy(rel_h.at[pl.ds(base_b, B_PER)], rel_v)
    pltpu.sync_copy(dR_h.at[pl.ds(base_b, B_PER)], dR)
    pltpu.sync_copy(dI_h.at[pl.ds(base_b, B_PER)], dI)
    pltpu.async_copy(tabE.at[sub_v], es, sem0).wait()
    pltpu.async_copy(tabR.at[rel_v], rr, sem0).wait()

    def prep(i, carry):
        for c in range(NC):
            s = pl.ds(c * NLANE, NLANE)
            si = pl.ds(D + c * NLANE, NLANE)
            er = es[i, s]
            ei = es[i, si]
            dr = dR[i, s]
            di = dI[i, s]
            aR[i, s] = er * dr - ei * di + rr[i, s]
            aI[i, s] = er * di + ei * dr + rr[i, si]
            ndR[i, s] = -dr
        return carry

    lax.fori_loop(0, B_PER, prep, 0)

    lane = lax.iota(jnp.int32, NLANE)
    bufs = ((rows.at[0], idx_v.at[0], semg0), (rows.at[1], idx_v.at[1], semg1))

    def issue(step, db):
        b_local = lax.shift_right_logical(step, 2)
        q = lax.bitwise_and(step, NQ - 1)
        rbuf, ibuf, sem = bufs[db]
        pltpu.sync_copy(ent_h.at[base_b + b_local, pl.ds(q * CHUNK, CHUNK)],
                        ibuf)
        pltpu.async_copy(tabE.at[ibuf], rbuf, sem)

    issue(0, 0)
    issue(1, 1)

    def compute(step, db):
        b_local = lax.shift_right_logical(step, 2)
        q = lax.bitwise_and(step, NQ - 1)
        rbuf, ibuf, sem = bufs[db]
        pltpu.make_async_copy(tabE.at[ibuf], rbuf, sem).wait()

        ars = [aR[b_local, pl.ds(c * NLANE, NLANE)] for c in range(NC)]
        ais = [aI[b_local, pl.ds(c * NLANE, NLANE)] for c in range(NC)]
        ndrs = [ndR[b_local, pl.ds(c * NLANE, NLANE)] for c in range(NC)]
        drs = [dR[b_local, pl.ds(c * NLANE, NLANE)] for c in range(NC)]
        dis = [dI[b_local, pl.ds(c * NLANE, NLANE)] for c in range(NC)]

        def group(h, gcarry):
            vec = jnp.zeros((NLANE,), jnp.float32)
            for jj in range(NLANE):
                j = h * NLANE + jj
                part = []
                for c in range(NC):
                    er = rbuf[j, pl.ds(c * NLANE, NLANE)]
                    ei = rbuf[j, pl.ds(D + c * NLANE, NLANE)]
                    t1 = er * ndrs[c] + ei * dis[c] + ars[c]
                    t2 = er * dis[c] + ei * drs[c] + ais[c]
                    part.append(jnp.abs(t1) + jnp.abs(t2))
                acc = (part[0] + part[1]) + (part[2] + part[3])
                vec = jnp.where(lane == jj,
                                jnp.full((NLANE,), jnp.sum(acc)), vec)
            scores[pl.ds(q * CHUNK + h * NLANE, NLANE)] = vec
            return gcarry

        lax.fori_loop(0, CHUNK // NLANE, group, 0)

        @pl.when(q == NQ - 1)
        def _():
            pltpu.sync_copy(scores, out_h.at[base_b + b_local])

    def step_pair(t, carry):
        for db in range(2):
            s = 2 * t + db
            compute(s, db)

            @pl.when(s + 2 < NSTEP)
            def _():
                issue(s + 2, db)

        return carry

    lax.fori_loop(0, NSTEP // 2, step_pair, 0)


def _sc_scores(tabE, tabR, sub, rel, ent, d_real, d_img):
    mesh = plsc.VectorSubcoreMesh(core_axis_name="c", subcore_axis_name="s")
    f = pl.kernel(
        _sc_body,
        out_type=jax.ShapeDtypeStruct((BS, NT), jnp.float32),
        mesh=mesh,
        compiler_params=pltpu.CompilerParams(
            needs_layout_passes=False, use_tc_tiling_on_sc=False),
        scratch_types=[
            pltpu.VMEM((B_PER,), jnp.int32),
            pltpu.VMEM((B_PER,), jnp.int32),
            pltpu.VMEM((B_PER, 2 * D), jnp.float32),
            pltpu.VMEM((B_PER, 2 * D), jnp.float32),
            pltpu.VMEM((B_PER, D), jnp.float32),
            pltpu.VMEM((B_PER, D), jnp.float32),
            pltpu.VMEM((B_PER, D), jnp.float32),
            pltpu.VMEM((B_PER, D), jnp.float32),
            pltpu.VMEM((B_PER, D), jnp.float32),
            pltpu.VMEM((2, CHUNK), jnp.int32),
            pltpu.VMEM((2, CHUNK, 2 * D), jnp.float32),
            pltpu.VMEM((NT,), jnp.float32),
            pltpu.VMEM((NLANE, NLANE), jnp.float32),
            pltpu.SemaphoreType.DMA,
            pltpu.SemaphoreType.DMA,
            pltpu.SemaphoreType.DMA,
        ],
    )
    return f(tabE, tabR, sub, rel, ent, d_real, d_img)


def _trig_body(day_ref, w1_ref, w2_ref, dr_ref, di_ref):
    day = day_ref[...]
    dr_ref[...] = jnp.cos(w2_ref[...] * day)
    di_ref[...] = jnp.sin(w1_ref[...] * day)


def _loss_body(x_ref, out_ref):
    x = x_ref[...]
    col = lax.broadcasted_iota(jnp.int32, (BS, NT), 1)
    valid = col < N_CAND
    xm = jnp.where(valid, x, -jnp.inf)
    m = jnp.max(xm, axis=1, keepdims=True)
    ssum = jnp.sum(jnp.where(valid, jnp.exp(xm - m), 0.0), axis=1,
                   keepdims=True)
    lse = m + jnp.log(ssum)
    pos = jnp.sum(jnp.where(col == 0, x, 0.0), axis=1, keepdims=True)
    val = jnp.sum(lse - pos) * (1.0 / BS)
    out_ref[...] = jnp.full((1, 1), val, jnp.float32)


def kernel(sub, rel, obj, year, month, day, neg, emb_E_real, emb_E_img,
           emb_R_real, emb_R_img, w1, w2):
    ent = jnp.concatenate([obj[:, None], neg], axis=1).astype(jnp.int32)
    ent = jnp.pad(ent, ((0, 0), (0, NT - N_CAND)))

    tabE = jnp.concatenate([emb_E_real, emb_E_img], axis=1)
    tabR = jnp.concatenate([emb_R_real, emb_R_img], axis=1)

    d_real, d_img = pl.pallas_call(
        _trig_body,
        out_shape=[
            jax.ShapeDtypeStruct((BS, D), jnp.float32),
            jax.ShapeDtypeStruct((BS, D), jnp.float32),
        ],
    )(day.reshape(BS, 1), w1.reshape(1, D), w2.reshape(1, D))

    scores = _sc_scores(tabE, tabR, sub.astype(jnp.int32),
                        rel.astype(jnp.int32), ent, d_real, d_img)

    loss = pl.pallas_call(
        _loss_body,
        out_shape=jax.ShapeDtypeStruct((1, 1), jnp.float32),
    )(scores)
    return loss[0, 0]
